# baseline (speedup 1.0000x reference)
import functools

import jax
import jax.numpy as jnp
from jax import lax
from jax.experimental import pallas as pl
from jax.experimental.pallas import tpu as pltpu
from jax.experimental.pallas import tpu_sc as plsc

DIM = 64
BATCH = 16384
HIST = 50
SUB = 128
NSUB_H = 4
N_C = HIST * NSUB_H


@functools.lru_cache(maxsize=None)
def _build():
    info = plsc.get_sparse_core_info()
    nc = info.num_cores
    per_w = BATCH // (nc * info.num_subcores)
    assert per_w == NSUB_H * SUB

    mesh = plsc.VectorSubcoreMesh(core_axis_name="c", subcore_axis_name="s")

    @functools.partial(
        pl.kernel,
        mesh=mesh,
        out_type=jax.ShapeDtypeStruct((HIST, DIM, BATCH), jnp.float32),
        scratch_types=[
            pltpu.VMEM((per_w,), jnp.int32),
            pltpu.VMEM((2, NSUB_H, SUB), jnp.int32),
            pltpu.VMEM((2, per_w), jnp.int32),
            pltpu.VMEM((SUB, 128), jnp.float32),
            pltpu.VMEM((SUB, 128), jnp.float32),
            pltpu.VMEM((DIM, SUB), jnp.float32),
            pltpu.VMEM((DIM, SUB), jnp.float32),
            pltpu.SemaphoreType.DMA,
            pltpu.SemaphoreType.DMA,
            pltpu.SemaphoreType.DMA,
            pltpu.SemaphoreType.DMA,
        ],
        compiler_params=pltpu.CompilerParams(
            use_tc_tiling_on_sc=True, needs_layout_passes=False),
    )
    def emb(tokt, table2, out3, tidx, widx, colb, rows0, rows1, tr0, tr1,
            sg0, sg1, so0, so1):
        wid = lax.axis_index("s") * nc + lax.axis_index("c")
        b0 = wid * per_w
        rows = (rows0, rows1)
        tr = (tr0, tr1)
        sg = (sg0, sg1)
        so = (so0, so1)
        iota16 = lax.iota(jnp.int32, 16)

        def prep(h, slot):
            pltpu.sync_copy(tokt.at[h, pl.ds(b0, per_w)], tidx)
            for j in range(NSUB_H):
                for k in range(SUB // 16):
                    t = tidx[pl.ds(j * SUB + k * 16, 16)]
                    widx[slot, j, pl.ds(k * 16, 16)] = (
                        lax.shift_right_logical(t, 1))
                    colb[slot, pl.ds(j * SUB + k * 16, 16)] = (
                        lax.shift_left(jnp.bitwise_and(t, 1), 6))

        def gather_start(c, b):
            slot = jnp.bitwise_and(c // NSUB_H, 1)
            j = jnp.remainder(c, NSUB_H)
            pltpu.async_copy(table2.at[widx.at[slot, j]], rows[b], sg[b])

        def gather_wait(b):
            pltpu.make_async_copy(table2.at[pl.ds(0, SUB)], rows[b],
                                  sg[b]).wait()

        def transpose(c, b):
            slot = jnp.bitwise_and(c // NSUB_H, 1)
            q = jnp.remainder(c, NSUB_H) * SUB
            for kb in range(SUB // 16):
                cb = colb[slot, pl.ds(q + kb * 16, 16)]
                rowv = kb * 16 + iota16

                def dbody(d, carry):
                    v = plsc.load_gather(rows[b], [rowv, cb + d])
                    tr[b].at[d][pl.ds(kb * 16, 16)] = v
                    return carry

                lax.fori_loop(0, DIM, dbody, 0, unroll=8)

        def out_start(c, b):
            h = c // NSUB_H
            bb = b0 + jnp.remainder(c, NSUB_H) * SUB
            pltpu.async_copy(tr[b], out3.at[h, :, pl.ds(bb, SUB)], so[b])

        def out_drain(b):
            pltpu.make_async_copy(tr[b], out3.at[0, :, pl.ds(0, SUB)],
                                  so[b]).wait()

        prep(0, 0)
        gather_start(0, 0)

        def body(step, carry):
            for u in (0, 1):
                c = 2 * step + u
                nb = 1 - u

                @pl.when(c + 1 < N_C)
                def _ahead():
                    @pl.when(jnp.remainder(c + 1, NSUB_H) == 0)
                    def _prep():
                        nh = (c + 1) // NSUB_H
                        prep(nh, jnp.bitwise_and(nh, 1))

                    gather_start(c + 1, nb)

                gather_wait(u)

                @pl.when(c >= 2)
                def _reuse():
                    out_drain(u)

                transpose(c, u)
                out_start(c, u)
            return carry

        lax.fori_loop(0, N_C // 2, body, 0)
        out_drain(0)
        out_drain(1)

    return emb


@jax.jit
def kernel(token_ids, weight):
    tokt = token_ids.T.astype(jnp.int32)
    table2 = weight.reshape(weight.shape[0] // 2, 2 * weight.shape[1])
    o3 = _build()(tokt, table2)
    return o3.transpose(2, 0, 1)


# speedup vs baseline: 1.3061x; 1.3061x over previous
import functools

import jax
import jax.numpy as jnp
from jax import lax
from jax.experimental import pallas as pl
from jax.experimental.pallas import tpu as pltpu
from jax.experimental.pallas import tpu_sc as plsc

DIM = 64
BATCH = 16384
HIST = 50
SUB = 256
NSUB_H = 2
N_C = HIST * NSUB_H


@functools.lru_cache(maxsize=None)
def _build():
    info = plsc.get_sparse_core_info()
    nc = info.num_cores
    per_w = BATCH // (nc * info.num_subcores)
    assert per_w == NSUB_H * SUB

    mesh = plsc.VectorSubcoreMesh(core_axis_name="c", subcore_axis_name="s")

    @functools.partial(
        pl.kernel,
        mesh=mesh,
        out_type=jax.ShapeDtypeStruct((HIST, DIM, BATCH), jnp.float32),
        scratch_types=[
            pltpu.VMEM((per_w,), jnp.int32),
            pltpu.VMEM((2, 2 * NSUB_H, 128), jnp.int32),
            pltpu.VMEM((2, per_w), jnp.int32),
            pltpu.VMEM((SUB, 128), jnp.float32),
            pltpu.VMEM((SUB, 128), jnp.float32),
            pltpu.VMEM((DIM, 128), jnp.float32),
            pltpu.VMEM((DIM, 128), jnp.float32),
            pltpu.SemaphoreType.DMA,
            pltpu.SemaphoreType.DMA,
            pltpu.SemaphoreType.DMA,
            pltpu.SemaphoreType.DMA,
        ],
        compiler_params=pltpu.CompilerParams(
            use_tc_tiling_on_sc=True, needs_layout_passes=False),
    )
    def emb(tokt, table2, out3, tidx, widx, colb, rows0, rows1, tr0, tr1,
            sg0, sg1, so0, so1):
        wid = lax.axis_index("s") * nc + lax.axis_index("c")
        b0 = wid * per_w
        rows = (rows0, rows1)
        tr = (tr0, tr1)
        sg = (sg0, sg1)
        so = (so0, so1)
        iota16 = lax.iota(jnp.int32, 16)

        def prep(h, slot):
            pltpu.sync_copy(tokt.at[h, pl.ds(b0, per_w)], tidx)
            for j in range(2 * NSUB_H):
                for k in range(8):
                    t = tidx[pl.ds(j * 128 + k * 16, 16)]
                    widx[slot, j, pl.ds(k * 16, 16)] = (
                        lax.shift_right_logical(t, 1))
                    colb[slot, pl.ds(j * 128 + k * 16, 16)] = (
                        lax.shift_left(jnp.bitwise_and(t, 1), 6))

        def gather_start(c, b):
            slot = jnp.bitwise_and(c // NSUB_H, 1)
            j = jnp.remainder(c, NSUB_H) * 2
            pltpu.async_copy(table2.at[widx.at[slot, j]],
                             rows[b].at[pl.ds(0, 128)], sg[b])
            pltpu.async_copy(table2.at[widx.at[slot, j + 1]],
                             rows[b].at[pl.ds(128, 128)], sg[b])

        def gather_wait(b):
            pltpu.make_async_copy(table2.at[pl.ds(0, SUB)], rows[b],
                                  sg[b]).wait()

        def transpose(c, b, half):
            slot = jnp.bitwise_and(c // NSUB_H, 1)
            q = jnp.remainder(c, NSUB_H) * SUB + half * 128
            for kb in range(8):
                cb = colb[slot, pl.ds(q + kb * 16, 16)]
                rowv = half * 128 + kb * 16 + iota16
                for dg in range(0, DIM, 8):
                    vs = [plsc.load_gather(rows[b], [rowv, cb + (dg + i)])
                          for i in range(8)]
                    for i in range(8):
                        tr[half].at[dg + i][pl.ds(kb * 16, 16)] = vs[i]

        def out_start(c, half):
            h = c // NSUB_H
            bb = b0 + jnp.remainder(c, NSUB_H) * SUB + half * 128
            pltpu.async_copy(tr[half], out3.at[h, :, pl.ds(bb, 128)],
                             so[half])

        def out_drain(half):
            pltpu.make_async_copy(tr[half], out3.at[0, :, pl.ds(0, 128)],
                                  so[half]).wait()

        prep(0, 0)
        gather_start(0, 0)

        def body(step, carry):
            for u in (0, 1):
                c = 2 * step + u
                nb = 1 - u

                @pl.when(c + 1 < N_C)
                def _ahead():
                    @pl.when(jnp.remainder(c + 1, NSUB_H) == 0)
                    def _prep():
                        nh = (c + 1) // NSUB_H
                        prep(nh, jnp.bitwise_and(nh, 1))

                    gather_start(c + 1, nb)

                gather_wait(u)
                for half in (0, 1):
                    @pl.when(c >= 1)
                    def _reuse(half=half):
                        out_drain(half)

                    transpose(c, u, half)
                    out_start(c, half)
            return carry

        lax.fori_loop(0, N_C // 2, body, 0)
        out_drain(0)
        out_drain(1)

    return emb


@jax.jit
def kernel(token_ids, weight):
    tokt = token_ids.T.astype(jnp.int32)
    table2 = weight.reshape(weight.shape[0] // 2, 2 * weight.shape[1])
    o3 = _build()(tokt, table2)
    return o3.transpose(2, 0, 1)


# speedup vs baseline: 1.3877x; 1.0625x over previous
import functools

import jax
import jax.numpy as jnp
from jax import lax
from jax.experimental import pallas as pl
from jax.experimental.pallas import tpu as pltpu
from jax.experimental.pallas import tpu_sc as plsc

DIM = 64
BATCH = 16384
HIST = 50
SUB = 256
NSUB_H = 2
N_C = HIST * NSUB_H


@functools.lru_cache(maxsize=None)
def _build():
    info = plsc.get_sparse_core_info()
    nc = info.num_cores
    per_w = BATCH // (nc * info.num_subcores)
    assert per_w == NSUB_H * SUB

    mesh = plsc.VectorSubcoreMesh(core_axis_name="c", subcore_axis_name="s")

    @functools.partial(
        pl.kernel,
        mesh=mesh,
        out_type=jax.ShapeDtypeStruct((HIST, DIM, BATCH), jnp.float32),
        scratch_types=[
            pltpu.VMEM((2, per_w), jnp.int32),
            pltpu.VMEM((2, 2 * NSUB_H, 128), jnp.int32),
            pltpu.VMEM((2, per_w), jnp.int32),
            pltpu.VMEM((SUB, 128), jnp.float32),
            pltpu.VMEM((SUB, 128), jnp.float32),
            pltpu.VMEM((DIM, 128), jnp.float32),
            pltpu.VMEM((DIM, 128), jnp.float32),
            pltpu.SemaphoreType.DMA,
            pltpu.SemaphoreType.DMA,
            pltpu.SemaphoreType.DMA,
            pltpu.SemaphoreType.DMA,
            pltpu.SemaphoreType.DMA,
        ],
        compiler_params=pltpu.CompilerParams(
            use_tc_tiling_on_sc=True, needs_layout_passes=False),
    )
    def emb(tokt, table2, out3, tidx, widx, colb, rows0, rows1, tr0, tr1,
            sg0, sg1, so0, so1, si):
        wid = lax.axis_index("s") * nc + lax.axis_index("c")
        b0 = wid * per_w
        rows = (rows0, rows1)
        tr = (tr0, tr1)
        sg = (sg0, sg1)
        so = (so0, so1)
        iota16 = lax.iota(jnp.int32, 16)

        def idx_start(h, slot):
            pltpu.async_copy(tokt.at[h, pl.ds(b0, per_w)], tidx.at[slot], si)

        def prep(h, slot):
            pltpu.make_async_copy(tokt.at[0, pl.ds(0, per_w)],
                                  tidx.at[slot], si).wait()
            for j in range(2 * NSUB_H):
                for k in range(8):
                    t = tidx[slot, pl.ds(j * 128 + k * 16, 16)]
                    widx[slot, j, pl.ds(k * 16, 16)] = (
                        lax.shift_right_logical(t, 1))
                    colb[slot, pl.ds(j * 128 + k * 16, 16)] = (
                        lax.shift_left(jnp.bitwise_and(t, 1), 6))

        def gather_start(c, b):
            slot = jnp.bitwise_and(c // NSUB_H, 1)
            j = jnp.remainder(c, NSUB_H) * 2
            pltpu.async_copy(table2.at[widx.at[slot, j]],
                             rows[b].at[pl.ds(0, 128)], sg[b])
            pltpu.async_copy(table2.at[widx.at[slot, j + 1]],
                             rows[b].at[pl.ds(128, 128)], sg[b])

        def gather_wait(b):
            pltpu.make_async_copy(table2.at[pl.ds(0, SUB)], rows[b],
                                  sg[b]).wait()

        def transpose(c, b, half):
            slot = jnp.bitwise_and(c // NSUB_H, 1)
            q = jnp.remainder(c, NSUB_H) * SUB + half * 128
            for kb in range(8):
                cb = colb[slot, pl.ds(q + kb * 16, 16)]
                rowv = half * 128 + kb * 16 + iota16
                for dg in range(0, DIM, 16):
                    vs = [plsc.load_gather(rows[b], [rowv, cb + (dg + i)])
                          for i in range(16)]
                    for i in range(16):
                        tr[half].at[dg + i][pl.ds(kb * 16, 16)] = vs[i]

        def out_start(c, half):
            h = c // NSUB_H
            bb = b0 + jnp.remainder(c, NSUB_H) * SUB + half * 128
            pltpu.async_copy(tr[half], out3.at[h, :, pl.ds(bb, 128)],
                             so[half])

        def out_drain(half):
            pltpu.make_async_copy(tr[half], out3.at[0, :, pl.ds(0, 128)],
                                  so[half]).wait()

        idx_start(0, 0)
        prep(0, 0)
        idx_start(1, 1)
        gather_start(0, 0)

        def body(step, carry):
            for u in (0, 1):
                c = 2 * step + u
                nb = 1 - u

                @pl.when(c + 1 < N_C)
                def _ahead():
                    @pl.when(jnp.remainder(c + 1, NSUB_H) == 0)
                    def _prep():
                        nh = (c + 1) // NSUB_H
                        prep(nh, jnp.bitwise_and(nh, 1))

                        @pl.when(nh + 1 < HIST)
                        def _istart():
                            idx_start(nh + 1, jnp.bitwise_and(nh + 1, 1))

                    gather_start(c + 1, nb)

                gather_wait(u)
                for half in (0, 1):
                    @pl.when(c >= 1)
                    def _reuse(half=half):
                        out_drain(half)

                    transpose(c, u, half)
                    out_start(c, half)
            return carry

        lax.fori_loop(0, N_C // 2, body, 0)
        out_drain(0)
        out_drain(1)

    return emb


@jax.jit
def kernel(token_ids, weight):
    tokt = token_ids.T.astype(jnp.int32)
    table2 = weight.reshape(weight.shape[0] // 2, 2 * weight.shape[1])
    o3 = _build()(tokt, table2)
    return o3.transpose(2, 0, 1)
